# Initial kernel scaffold; baseline (speedup 1.0000x reference)
import jax
import jax.numpy as jnp
from jax.experimental import pallas as pl


def kernel(x, edge_index, edge_weight, batch, W1, b1, g1, be1, W2, b2, g2, be2, fcW, fcb):
    raise NotImplementedError("write your pallas kernel here")


# speedup vs baseline: 7.1049x; 7.1049x over previous
import jax
import jax.numpy as jnp
from jax import lax
from jax.experimental import pallas as pl
from jax.experimental.pallas import tpu as pltpu
from jax.experimental.pallas import tpu_sc as plsc

N_NODES = 10000
N_PAD = 10240
N_EDGES = 160000
HALF = 128
CHUNK = 128
N_CHUNKS = N_EDGES // CHUNK
N_SUB = 16
N_CORES = 2
N_WORKERS = N_SUB * N_CORES
ROWS_PER_SUB = N_PAD // N_SUB
ZROWS = 64
N_GRAPHS = 64
ROW_BLK = 2000


def _sc_mesh():
    return plsc.VectorSubcoreMesh(core_axis_name="c", subcore_axis_name="s")


_SC_PARAMS = pltpu.CompilerParams(needs_layout_passes=False)


def _sc_deg(col, ew):
    def body(col_hbm, ew_hbm, deg_hbm, colv, eww, zbuf, acc):
        c = lax.axis_index("c")
        s = lax.axis_index("s")

        @pl.when(c == 0)
        def _():
            def zb(i, _):
                zbuf[pl.ds(i * 16, 16)] = jnp.zeros((16,), jnp.float32)
                return 0
            lax.fori_loop(0, ROWS_PER_SUB // 16, zb, 0)
            pltpu.sync_copy(zbuf, acc.at[pl.ds(s * ROWS_PER_SUB, ROWS_PER_SUB)])
            plsc.subcore_barrier()

            n_i = (N_CHUNKS - s + N_SUB - 1) // N_SUB

            def step(i, _):
                e0 = (s + i * N_SUB) * CHUNK
                pltpu.sync_copy(col_hbm.at[pl.ds(e0, CHUNK)], colv)
                pltpu.sync_copy(ew_hbm.at[pl.ds(e0, CHUNK)], eww)
                pltpu.sync_copy(eww, acc.at[colv], add=True)
                return 0
            lax.fori_loop(0, n_i, step, 0)
            plsc.subcore_barrier()
            pltpu.sync_copy(acc.at[pl.ds(s * ROWS_PER_SUB, ROWS_PER_SUB)],
                            deg_hbm.at[pl.ds(s * ROWS_PER_SUB, ROWS_PER_SUB)])

    f = pl.kernel(
        body,
        out_type=jax.ShapeDtypeStruct((N_PAD,), jnp.float32),
        mesh=_sc_mesh(),
        compiler_params=_SC_PARAMS,
        scratch_types=[
            pltpu.VMEM((CHUNK,), jnp.int32),
            pltpu.VMEM((CHUNK,), jnp.float32),
            pltpu.VMEM((ROWS_PER_SUB,), jnp.float32),
            pltpu.VMEM_SHARED((N_PAD,), jnp.float32),
        ],
    )
    return f(col, ew)


def _sc_agg(row, col, ew, dis_pad, xcat):
    def body(row_hbm, col_hbm, ew_hbm, dis_hbm, x_hbm, out_hbm,
             rowv, colv, gidx, eww, normv, disv, rows_v, zbuf, acc, sem):
        c = lax.axis_index("c")
        s = lax.axis_index("s")
        base_feat = c * N_PAD

        pltpu.sync_copy(dis_hbm, disv)

        def zb(r, _):
            for l in range(HALF // 16):
                zbuf[r, pl.ds(l * 16, 16)] = jnp.zeros((16,), jnp.float32)
            return 0
        lax.fori_loop(0, ZROWS, zb, 0)

        def zc(t, _):
            pltpu.sync_copy(
                zbuf, acc.at[pl.ds(s * ROWS_PER_SUB + t * ZROWS, ZROWS)])
            return 0
        lax.fori_loop(0, ROWS_PER_SUB // ZROWS, zc, 0)
        plsc.subcore_barrier()

        n_i = (N_CHUNKS - s + N_SUB - 1) // N_SUB

        def step(i, _):
            e0 = (s + i * N_SUB) * CHUNK
            pltpu.sync_copy(row_hbm.at[pl.ds(e0, CHUNK)], rowv)
            pltpu.sync_copy(col_hbm.at[pl.ds(e0, CHUNK)], colv)
            pltpu.sync_copy(ew_hbm.at[pl.ds(e0, CHUNK)], eww)
            for j in range(CHUNK // 16):
                sl = pl.ds(j * 16, 16)
                r16 = rowv[sl]
                c16 = colv[sl]
                dr = plsc.load_gather(disv, [r16])
                dc = plsc.load_gather(disv, [c16])
                normv[sl] = dr * dc * eww[sl]
                gidx[sl] = r16 + base_feat
            pltpu.async_copy(x_hbm.at[gidx], rows_v, sem).wait()

            def scale(jj, _):
                jjv = jnp.full((16,), jj, jnp.int32)
                sn = plsc.load_gather(normv, [jjv])
                for l in range(HALF // 16):
                    sl = pl.ds(l * 16, 16)
                    rows_v[jj, sl] = rows_v[jj, sl] * sn
                return 0
            lax.fori_loop(0, CHUNK, scale, 0)
            pltpu.sync_copy(rows_v, acc.at[colv], add=True)
            return 0
        lax.fori_loop(0, n_i, step, 0)
        plsc.subcore_barrier()

        out_base = base_feat + s * ROWS_PER_SUB
        pltpu.sync_copy(acc.at[pl.ds(s * ROWS_PER_SUB, ROWS_PER_SUB)],
                        out_hbm.at[pl.ds(out_base, ROWS_PER_SUB)])

    f = pl.kernel(
        body,
        out_type=jax.ShapeDtypeStruct((2 * N_PAD, HALF), jnp.float32),
        mesh=_sc_mesh(),
        compiler_params=_SC_PARAMS,
        scratch_types=[
            pltpu.VMEM((CHUNK,), jnp.int32),
            pltpu.VMEM((CHUNK,), jnp.int32),
            pltpu.VMEM((CHUNK,), jnp.int32),
            pltpu.VMEM((CHUNK,), jnp.float32),
            pltpu.VMEM((CHUNK,), jnp.float32),
            pltpu.VMEM((N_PAD,), jnp.float32),
            pltpu.VMEM((CHUNK, HALF), jnp.float32),
            pltpu.VMEM((ZROWS, HALF), jnp.float32),
            pltpu.VMEM_SHARED((N_PAD, HALF), jnp.float32),
            pltpu.SemaphoreType.DMA,
        ],
    )
    return f(row, col, ew, dis_pad, xcat)


def _tc_layer1(agg1, x, dis_col, W1, b1r, g1r, be1r, W2):
    def body(agg_ref, x_ref, dis_ref, W1_ref, b1_ref, g1_ref, be1_ref,
             W2_ref, t2_ref):
        d = dis_ref[...]
        pre = agg_ref[...] + d * d * x_ref[...]
        h = jnp.dot(pre, W1_ref[...], preferred_element_type=jnp.float32,
                    precision=lax.Precision.HIGHEST) + b1_ref[...]
        h = jnp.maximum(h, 0.0)
        mu = jnp.mean(h, axis=-1, keepdims=True)
        xc = h - mu
        var = jnp.mean(xc * xc, axis=-1, keepdims=True)
        h = xc * lax.rsqrt(var + 1e-5) * g1_ref[...] + be1_ref[...]
        t2_ref[...] = jnp.dot(h, W2_ref[...], preferred_element_type=jnp.float32,
                              precision=lax.Precision.HIGHEST)

    grid = N_NODES // ROW_BLK
    return pl.pallas_call(
        body,
        grid=(grid,),
        in_specs=[
            pl.BlockSpec((ROW_BLK, 256), lambda i: (i, 0)),
            pl.BlockSpec((ROW_BLK, 256), lambda i: (i, 0)),
            pl.BlockSpec((ROW_BLK, 1), lambda i: (i, 0)),
            pl.BlockSpec((256, 512), lambda i: (0, 0)),
            pl.BlockSpec((1, 512), lambda i: (0, 0)),
            pl.BlockSpec((1, 512), lambda i: (0, 0)),
            pl.BlockSpec((1, 512), lambda i: (0, 0)),
            pl.BlockSpec((512, 256), lambda i: (0, 0)),
        ],
        out_specs=pl.BlockSpec((ROW_BLK, 256), lambda i: (i, 0)),
        out_shape=jax.ShapeDtypeStruct((N_NODES, 256), jnp.float32),
    )(agg1, x, dis_col, W1, b1r, g1r, be1r, W2)


def _tc_layer2_pool(agg2, t2, dis_col, b2r, g2r, be2r, batch_row, fcW, fcb_r):
    def body(agg_ref, t2_ref, dis_ref, b2_ref, g2_ref, be2_ref, bat_ref,
             fcW_ref, fcb_ref, out_ref, pooled, cnt):
        i = pl.program_id(0)

        @pl.when(i == 0)
        def _():
            pooled[...] = jnp.zeros_like(pooled)
            cnt[...] = jnp.zeros_like(cnt)

        d = dis_ref[...]
        pre = agg_ref[...] + d * d * t2_ref[...] + b2_ref[...]
        h = jnp.maximum(pre, 0.0)
        mu = jnp.mean(h, axis=-1, keepdims=True)
        xc = h - mu
        var = jnp.mean(xc * xc, axis=-1, keepdims=True)
        h = xc * lax.rsqrt(var + 1e-5) * g2_ref[...] + be2_ref[...]

        seg = bat_ref[...].reshape(1, ROW_BLK)
        gids = lax.broadcasted_iota(jnp.int32, (N_GRAPHS, ROW_BLK), 0)
        oh = (gids == seg).astype(jnp.float32)
        pooled[...] += jnp.dot(oh, h, preferred_element_type=jnp.float32,
                               precision=lax.Precision.HIGHEST)
        cnt[...] += jnp.sum(oh, axis=1, keepdims=True)

        @pl.when(i == pl.num_programs(0) - 1)
        def _():
            pm = pooled[...] / jnp.maximum(cnt[...], 1.0)
            out_ref[...] = jnp.dot(pm, fcW_ref[...],
                                   preferred_element_type=jnp.float32,
                                   precision=lax.Precision.HIGHEST) + fcb_ref[...]

    grid = N_NODES // ROW_BLK
    return pl.pallas_call(
        body,
        grid=(grid,),
        in_specs=[
            pl.BlockSpec((ROW_BLK, 256), lambda i: (i, 0)),
            pl.BlockSpec((ROW_BLK, 256), lambda i: (i, 0)),
            pl.BlockSpec((ROW_BLK, 1), lambda i: (i, 0)),
            pl.BlockSpec((1, 256), lambda i: (0, 0)),
            pl.BlockSpec((1, 256), lambda i: (0, 0)),
            pl.BlockSpec((1, 256), lambda i: (0, 0)),
            pl.BlockSpec((1, 1, ROW_BLK), lambda i: (i, 0, 0)),
            pl.BlockSpec((256, 128), lambda i: (0, 0)),
            pl.BlockSpec((1, 128), lambda i: (0, 0)),
        ],
        out_specs=pl.BlockSpec((N_GRAPHS, 128), lambda i: (0, 0)),
        out_shape=jax.ShapeDtypeStruct((N_GRAPHS, 128), jnp.float32),
        scratch_shapes=[
            pltpu.VMEM((N_GRAPHS, 256), jnp.float32),
            pltpu.VMEM((N_GRAPHS, 1), jnp.float32),
        ],
    )(agg2, t2, dis_col, b2r, g2r, be2r, batch_row, fcW, fcb_r)


def _split_halves(a):
    ap = jnp.pad(a, ((0, N_PAD - N_NODES), (0, 0)))
    return ap.reshape(N_PAD, 2, HALF).transpose(1, 0, 2).reshape(2 * N_PAD, HALF)


def _merge_halves(a):
    return (a.reshape(2, N_PAD, HALF).transpose(1, 0, 2)
            .reshape(N_PAD, 256)[:N_NODES])


def kernel(x, edge_index, edge_weight, batch,
           W1, b1, g1, be1, W2, b2, g2, be2, fcW, fcb):
    row = edge_index[0].astype(jnp.int32)
    col = edge_index[1].astype(jnp.int32)
    ew = edge_weight.astype(jnp.float32)

    deg_sc = _sc_deg(col, ew)
    deg = deg_sc[:N_NODES] + 1.0
    dis = jnp.where(deg > 0, lax.rsqrt(deg), 0.0)
    dis_pad = jnp.pad(dis, (0, N_PAD - N_NODES))
    dis_col = dis[:, None]

    agg1 = _merge_halves(_sc_agg(row, col, ew, dis_pad, _split_halves(x)))
    t2 = _tc_layer1(agg1, x, dis_col, W1, b1[None], g1[None], be1[None], W2)
    agg2 = _merge_halves(_sc_agg(row, col, ew, dis_pad, _split_halves(t2)))
    return _tc_layer2_pool(agg2, t2, dis_col, b2[None], g2[None], be2[None],
                           batch.astype(jnp.int32).reshape(
                               N_NODES // ROW_BLK, 1, ROW_BLK),
                           fcW, fcb[None])


# speedup vs baseline: 14.3666x; 2.0221x over previous
import jax
import jax.numpy as jnp
from jax import lax
from jax.experimental import pallas as pl
from jax.experimental.pallas import tpu as pltpu
from jax.experimental.pallas import tpu_sc as plsc

N_NODES = 10000
N_PAD = 10240
N_EDGES = 160000
HALF = 128
N_SUB = 16
N_CORES = 2
DCHUNK = 80
DCPW = (N_EDGES // DCHUNK) // N_SUB
ACHUNK = 64
N_CH = N_EDGES // ACHUNK
RING = 4
ROWS_PER_SUB = N_PAD // N_SUB
ZROWS = 16
N_GRAPHS = 64
ROW_BLK = 2000


def _sc_mesh():
    return plsc.VectorSubcoreMesh(core_axis_name="c", subcore_axis_name="s")


_SC_PARAMS = pltpu.CompilerParams(needs_layout_passes=False)


def _sc_deg(col2, ew2):
    def body(col_hbm, ew_hbm, deg_hbm, colv2, eww2, zbuf, acc, sem):
        c = lax.axis_index("c")
        s = lax.axis_index("s")

        @pl.when(c == 0)
        def _():
            def zb(i, _):
                zbuf[pl.ds(i * 16, 16)] = jnp.zeros((16,), jnp.float32)
                return 0
            lax.fori_loop(0, ROWS_PER_SUB // 16, zb, 0)
            pltpu.sync_copy(zbuf, acc.at[pl.ds(s * ROWS_PER_SUB, ROWS_PER_SUB)])
            plsc.subcore_barrier()

            pltpu.sync_copy(col_hbm.at[s], colv2)
            pltpu.sync_copy(ew_hbm.at[s], eww2)

            def step(t, _):
                for b in range(5):
                    ci = t * 5 + b
                    pltpu.make_async_copy(
                        eww2.at[ci], acc.at[colv2.at[ci]], sem).start(add=True)
                for b in range(5):
                    pltpu.make_async_copy(
                        eww2.at[0], acc.at[colv2.at[0]], sem).wait()
                return 0
            lax.fori_loop(0, DCPW // 5, step, 0)
            plsc.subcore_barrier()
            pltpu.sync_copy(acc.at[pl.ds(s * ROWS_PER_SUB, ROWS_PER_SUB)],
                            deg_hbm.at[pl.ds(s * ROWS_PER_SUB, ROWS_PER_SUB)])

    f = pl.kernel(
        body,
        out_type=jax.ShapeDtypeStruct((N_PAD,), jnp.float32),
        mesh=_sc_mesh(),
        compiler_params=_SC_PARAMS,
        scratch_types=[
            pltpu.VMEM((DCPW, DCHUNK), jnp.int32),
            pltpu.VMEM((DCPW, DCHUNK), jnp.float32),
            pltpu.VMEM((ROWS_PER_SUB,), jnp.float32),
            pltpu.VMEM_SHARED((N_PAD,), jnp.float32),
            pltpu.SemaphoreType.DMA,
        ],
    )
    return f(col2, ew2)


def _sc_agg(rce, dis_pad, xcat):
    def body(rce_hbm, dis_hbm, x_hbm, out_hbm,
             disv,
             rce0, rce1, rce2, rce3,
             colb0, colb1, colb2, colb3,
             gidx0, gidx1, gidx2, gidx3,
             norm0, norm1, norm2, norm3,
             rows0, rows1, rows2, rows3,
             zbuf, acc,
             seme0, seme1, seme2, seme3,
             semg0, semg1, semg2, semg3,
             sems0, sems1, sems2, sems3):
        c = lax.axis_index("c")
        s = lax.axis_index("s")
        base_feat = c * N_PAD
        rceb = [rce0, rce1, rce2, rce3]
        colb = [colb0, colb1, colb2, colb3]
        gidx = [gidx0, gidx1, gidx2, gidx3]
        normb = [norm0, norm1, norm2, norm3]
        rowsb = [rows0, rows1, rows2, rows3]
        seme = [seme0, seme1, seme2, seme3]
        semg = [semg0, semg1, semg2, semg3]
        sems = [sems0, sems1, sems2, sems3]

        pltpu.sync_copy(dis_hbm, disv)

        def zb(r, _):
            for l in range(HALF // 16):
                zbuf[r, pl.ds(l * 16, 16)] = jnp.zeros((16,), jnp.float32)
            return 0
        lax.fori_loop(0, ZROWS, zb, 0)

        def zc(t, _):
            pltpu.sync_copy(
                zbuf, acc.at[pl.ds(s * ROWS_PER_SUB + t * ZROWS, ZROWS)])
            return 0
        lax.fori_loop(0, ROWS_PER_SUB // ZROWS, zc, 0)
        plsc.subcore_barrier()

        n_w = (N_CH - s + N_SUB - 1) // N_SUB

        def start_edge(b, i):
            g = s + N_SUB * i
            pltpu.make_async_copy(
                rce_hbm.at[pl.ds(g * 3 * ACHUNK, 3 * ACHUNK)],
                rceb[b], seme[b]).start()

        def wait_edge(b):
            pltpu.make_async_copy(
                rce_hbm.at[pl.ds(0, 3 * ACHUNK)], rceb[b], seme[b]).wait()

        def build(b):
            for j in range(ACHUNK // 16):
                sl = pl.ds(j * 16, 16)
                r16 = rceb[b][sl]
                c16 = rceb[b][pl.ds(ACHUNK + j * 16, 16)]
                w16 = plsc.bitcast(
                    rceb[b][pl.ds(2 * ACHUNK + j * 16, 16)], jnp.float32)
                dr = plsc.load_gather(disv, [r16])
                dc = plsc.load_gather(disv, [c16])
                normb[b][sl] = dr * dc * w16
                gidx[b][sl] = r16 + base_feat
                colb[b][sl] = c16

        def start_gather(b):
            pltpu.make_async_copy(x_hbm.at[gidx[b]], rowsb[b], semg[b]).start()

        def wait_gather(b):
            pltpu.make_async_copy(x_hbm.at[gidx[b]], rowsb[b], semg[b]).wait()

        def start_scatter(b):
            pltpu.make_async_copy(
                rowsb[b], acc.at[colb[b]], sems[b]).start(add=True)

        def wait_scatter(b):
            pltpu.make_async_copy(
                rowsb[b], acc.at[colb[b]], sems[b]).wait()

        for b in range(RING):
            start_edge(b, b)
        for b in range(2):
            wait_edge(b)
            build(b)
            start_gather(b)

        def visit(b, i):
            @pl.when(i < n_w)
            def _():
                wait_gather(b)

                def scale(jj, _):
                    jjv = jnp.full((16,), jj, jnp.int32)
                    sn = plsc.load_gather(normb[b], [jjv])
                    for l in range(HALF // 16):
                        sl = pl.ds(l * 16, 16)
                        rowsb[b][jj, sl] = rowsb[b][jj, sl] * sn
                    return 0
                lax.fori_loop(0, ACHUNK, scale, 0)
                start_scatter(b)
                b2 = (b + 2) % RING

                @pl.when(i + 2 < n_w)
                def _():
                    @pl.when(i >= 2)
                    def _():
                        wait_scatter(b2)
                    wait_edge(b2)
                    build(b2)
                    start_gather(b2)

                @pl.when(i + RING < n_w)
                def _():
                    start_edge(b, i + RING)

        def tbody(t, _):
            for b in range(RING):
                visit(b, t * RING + b)
            return 0
        max_nw = (N_CH + N_SUB - 1) // N_SUB
        lax.fori_loop(0, (max_nw + RING - 1) // RING, tbody, 0)
        for b in range(RING):
            wait_scatter(b)
        plsc.subcore_barrier()

        out_base = base_feat + s * ROWS_PER_SUB
        pltpu.sync_copy(acc.at[pl.ds(s * ROWS_PER_SUB, ROWS_PER_SUB)],
                        out_hbm.at[pl.ds(out_base, ROWS_PER_SUB)])

    f = pl.kernel(
        body,
        out_type=jax.ShapeDtypeStruct((2 * N_PAD, HALF), jnp.float32),
        mesh=_sc_mesh(),
        compiler_params=_SC_PARAMS,
        scratch_types=[
            pltpu.VMEM((N_PAD,), jnp.float32),
        ] + [pltpu.VMEM((3 * ACHUNK,), jnp.int32) for _ in range(RING)]
          + [pltpu.VMEM((ACHUNK,), jnp.int32) for _ in range(RING)]
          + [pltpu.VMEM((ACHUNK,), jnp.int32) for _ in range(RING)]
          + [pltpu.VMEM((ACHUNK,), jnp.float32) for _ in range(RING)]
          + [pltpu.VMEM((ACHUNK, HALF), jnp.float32) for _ in range(RING)]
          + [
            pltpu.VMEM((ZROWS, HALF), jnp.float32),
            pltpu.VMEM_SHARED((N_PAD, HALF), jnp.float32),
        ] + [pltpu.SemaphoreType.DMA for _ in range(3 * RING)],
    )
    return f(rce, dis_pad, xcat)


def _tc_layer1(agg1, x, dis_col, W1, b1r, g1r, be1r, W2):
    def body(agg_ref, x_ref, dis_ref, W1_ref, b1_ref, g1_ref, be1_ref,
             W2_ref, t2_ref):
        d = dis_ref[...]
        pre = agg_ref[...] + d * d * x_ref[...]
        h = jnp.dot(pre, W1_ref[...], preferred_element_type=jnp.float32,
                    precision=lax.Precision.HIGHEST) + b1_ref[...]
        h = jnp.maximum(h, 0.0)
        mu = jnp.mean(h, axis=-1, keepdims=True)
        xc = h - mu
        var = jnp.mean(xc * xc, axis=-1, keepdims=True)
        h = xc * lax.rsqrt(var + 1e-5) * g1_ref[...] + be1_ref[...]
        t2_ref[...] = jnp.dot(h, W2_ref[...], preferred_element_type=jnp.float32,
                              precision=lax.Precision.HIGHEST)

    grid = N_NODES // ROW_BLK
    return pl.pallas_call(
        body,
        grid=(grid,),
        in_specs=[
            pl.BlockSpec((ROW_BLK, 256), lambda i: (i, 0)),
            pl.BlockSpec((ROW_BLK, 256), lambda i: (i, 0)),
            pl.BlockSpec((ROW_BLK, 1), lambda i: (i, 0)),
            pl.BlockSpec((256, 512), lambda i: (0, 0)),
            pl.BlockSpec((1, 512), lambda i: (0, 0)),
            pl.BlockSpec((1, 512), lambda i: (0, 0)),
            pl.BlockSpec((1, 512), lambda i: (0, 0)),
            pl.BlockSpec((512, 256), lambda i: (0, 0)),
        ],
        out_specs=pl.BlockSpec((ROW_BLK, 256), lambda i: (i, 0)),
        out_shape=jax.ShapeDtypeStruct((N_NODES, 256), jnp.float32),
    )(agg1, x, dis_col, W1, b1r, g1r, be1r, W2)


def _tc_layer2_pool(agg2, t2, dis_col, b2r, g2r, be2r, batch_row, fcW, fcb_r):
    def body(agg_ref, t2_ref, dis_ref, b2_ref, g2_ref, be2_ref, bat_ref,
             fcW_ref, fcb_ref, out_ref, pooled, cnt):
        i = pl.program_id(0)

        @pl.when(i == 0)
        def _():
            pooled[...] = jnp.zeros_like(pooled)
            cnt[...] = jnp.zeros_like(cnt)

        d = dis_ref[...]
        pre = agg_ref[...] + d * d * t2_ref[...] + b2_ref[...]
        h = jnp.maximum(pre, 0.0)
        mu = jnp.mean(h, axis=-1, keepdims=True)
        xc = h - mu
        var = jnp.mean(xc * xc, axis=-1, keepdims=True)
        h = xc * lax.rsqrt(var + 1e-5) * g2_ref[...] + be2_ref[...]

        seg = bat_ref[...].reshape(1, ROW_BLK)
        gids = lax.broadcasted_iota(jnp.int32, (N_GRAPHS, ROW_BLK), 0)
        oh = (gids == seg).astype(jnp.float32)
        pooled[...] += jnp.dot(oh, h, preferred_element_type=jnp.float32,
                               precision=lax.Precision.HIGHEST)
        cnt[...] += jnp.sum(oh, axis=1, keepdims=True)

        @pl.when(i == pl.num_programs(0) - 1)
        def _():
            pm = pooled[...] / jnp.maximum(cnt[...], 1.0)
            out_ref[...] = jnp.dot(pm, fcW_ref[...],
                                   preferred_element_type=jnp.float32,
                                   precision=lax.Precision.HIGHEST) + fcb_ref[...]

    grid = N_NODES // ROW_BLK
    return pl.pallas_call(
        body,
        grid=(grid,),
        in_specs=[
            pl.BlockSpec((ROW_BLK, 256), lambda i: (i, 0)),
            pl.BlockSpec((ROW_BLK, 256), lambda i: (i, 0)),
            pl.BlockSpec((ROW_BLK, 1), lambda i: (i, 0)),
            pl.BlockSpec((1, 256), lambda i: (0, 0)),
            pl.BlockSpec((1, 256), lambda i: (0, 0)),
            pl.BlockSpec((1, 256), lambda i: (0, 0)),
            pl.BlockSpec((1, 1, ROW_BLK), lambda i: (i, 0, 0)),
            pl.BlockSpec((256, 128), lambda i: (0, 0)),
            pl.BlockSpec((1, 128), lambda i: (0, 0)),
        ],
        out_specs=pl.BlockSpec((N_GRAPHS, 128), lambda i: (0, 0)),
        out_shape=jax.ShapeDtypeStruct((N_GRAPHS, 128), jnp.float32),
        scratch_shapes=[
            pltpu.VMEM((N_GRAPHS, 256), jnp.float32),
            pltpu.VMEM((N_GRAPHS, 1), jnp.float32),
        ],
    )(agg2, t2, dis_col, b2r, g2r, be2r, batch_row, fcW, fcb_r)


def _split_halves(a):
    ap = jnp.pad(a, ((0, N_PAD - N_NODES), (0, 0)))
    return ap.reshape(N_PAD, 2, HALF).transpose(1, 0, 2).reshape(2 * N_PAD, HALF)


def _merge_halves(a):
    return (a.reshape(2, N_PAD, HALF).transpose(1, 0, 2)
            .reshape(N_PAD, 256)[:N_NODES])


def kernel(x, edge_index, edge_weight, batch,
           W1, b1, g1, be1, W2, b2, g2, be2, fcW, fcb):
    row = edge_index[0].astype(jnp.int32)
    col = edge_index[1].astype(jnp.int32)
    ew = edge_weight.astype(jnp.float32)
    rce = jnp.concatenate(
        [row.reshape(N_CH, ACHUNK), col.reshape(N_CH, ACHUNK),
         lax.bitcast_convert_type(ew, jnp.int32).reshape(N_CH, ACHUNK)],
        axis=1).reshape(-1)

    deg_sc = _sc_deg(col.reshape(N_SUB, DCPW, DCHUNK),
                     ew.reshape(N_SUB, DCPW, DCHUNK))
    deg = deg_sc[:N_NODES] + 1.0
    dis = jnp.where(deg > 0, lax.rsqrt(deg), 0.0)
    dis_pad = jnp.pad(dis, (0, N_PAD - N_NODES))
    dis_col = dis[:, None]

    agg1 = _merge_halves(_sc_agg(rce, dis_pad, _split_halves(x)))
    t2 = _tc_layer1(agg1, x, dis_col, W1, b1[None], g1[None], be1[None], W2)
    agg2 = _merge_halves(_sc_agg(rce, dis_pad, _split_halves(t2)))
    return _tc_layer2_pool(agg2, t2, dis_col, b2[None], g2[None], be2[None],
                           batch.astype(jnp.int32).reshape(
                               N_NODES // ROW_BLK, 1, ROW_BLK),
                           fcW, fcb[None])


# speedup vs baseline: 15.6550x; 1.0897x over previous
import jax
import jax.numpy as jnp
from jax import lax
from jax.experimental import pallas as pl
from jax.experimental.pallas import tpu as pltpu
from jax.experimental.pallas import tpu_sc as plsc

N_NODES = 10000
N_PAD = 10240
N_EDGES = 160000
HALF = 128
N_SUB = 16
N_CORES = 2
DCHUNK = 80
DCPW = (N_EDGES // DCHUNK) // N_SUB
ACHUNK = 64
N_CH = N_EDGES // ACHUNK
RING = 4
ROWS_PER_SUB = N_PAD // N_SUB
ZROWS = 16
N_GRAPHS = 64
ROW_BLK = 2000


def _sc_mesh():
    return plsc.VectorSubcoreMesh(core_axis_name="c", subcore_axis_name="s")


_SC_PARAMS = pltpu.CompilerParams(needs_layout_passes=False)


def _sc_deg(col2, ew2):
    def body(col_hbm, ew_hbm, deg_hbm, colv2, eww2, zbuf, acc, sem):
        c = lax.axis_index("c")
        s = lax.axis_index("s")

        @pl.when(c == 0)
        def _():
            def zb(i, _):
                zbuf[pl.ds(i * 16, 16)] = jnp.zeros((16,), jnp.float32)
                return 0
            lax.fori_loop(0, ROWS_PER_SUB // 16, zb, 0)
            pltpu.sync_copy(zbuf, acc.at[pl.ds(s * ROWS_PER_SUB, ROWS_PER_SUB)])
            plsc.subcore_barrier()

            pltpu.sync_copy(col_hbm.at[s], colv2)
            pltpu.sync_copy(ew_hbm.at[s], eww2)

            def step(t, _):
                for b in range(5):
                    ci = t * 5 + b
                    pltpu.make_async_copy(
                        eww2.at[ci], acc.at[colv2.at[ci]], sem).start(add=True)
                for b in range(5):
                    pltpu.make_async_copy(
                        eww2.at[0], acc.at[colv2.at[0]], sem).wait()
                return 0
            lax.fori_loop(0, DCPW // 5, step, 0)
            plsc.subcore_barrier()
            pltpu.sync_copy(acc.at[pl.ds(s * ROWS_PER_SUB, ROWS_PER_SUB)],
                            deg_hbm.at[pl.ds(s * ROWS_PER_SUB, ROWS_PER_SUB)])

    f = pl.kernel(
        body,
        out_type=jax.ShapeDtypeStruct((N_PAD,), jnp.float32),
        mesh=_sc_mesh(),
        compiler_params=_SC_PARAMS,
        scratch_types=[
            pltpu.VMEM((DCPW, DCHUNK), jnp.int32),
            pltpu.VMEM((DCPW, DCHUNK), jnp.float32),
            pltpu.VMEM((ROWS_PER_SUB,), jnp.float32),
            pltpu.VMEM_SHARED((N_PAD,), jnp.float32),
            pltpu.SemaphoreType.DMA,
        ],
    )
    return f(col2, ew2)


def _sc_agg(rce, dis_pad, xcat):
    def body(rce_hbm, dis_hbm, x_hbm, out_hbm,
             disv,
             rce0, rce1, rce2, rce3,
             colb0, colb1, colb2, colb3,
             gidx0, gidx1, gidx2, gidx3,
             norm0, norm1, norm2, norm3,
             rows0, rows1, rows2, rows3,
             zbuf, acc,
             seme0, seme1, seme2, seme3,
             semg0, semg1, semg2, semg3,
             sems0, sems1, sems2, sems3):
        c = lax.axis_index("c")
        s = lax.axis_index("s")
        base_feat = c * N_PAD
        rceb = [rce0, rce1, rce2, rce3]
        colb = [colb0, colb1, colb2, colb3]
        gidx = [gidx0, gidx1, gidx2, gidx3]
        normb = [norm0, norm1, norm2, norm3]
        rowsb = [rows0, rows1, rows2, rows3]
        seme = [seme0, seme1, seme2, seme3]
        semg = [semg0, semg1, semg2, semg3]
        sems = [sems0, sems1, sems2, sems3]

        pltpu.sync_copy(dis_hbm, disv)

        def zb(r, _):
            for l in range(HALF // 16):
                zbuf[r, pl.ds(l * 16, 16)] = jnp.zeros((16,), jnp.float32)
            return 0
        lax.fori_loop(0, ZROWS, zb, 0)

        def zc(t, _):
            pltpu.sync_copy(
                zbuf, acc.at[pl.ds(s * ROWS_PER_SUB + t * ZROWS, ZROWS)])
            return 0
        lax.fori_loop(0, ROWS_PER_SUB // ZROWS, zc, 0)
        plsc.subcore_barrier()

        n_w = (N_CH - s + N_SUB - 1) // N_SUB

        def start_edge(b, i):
            g = s + N_SUB * i
            pltpu.make_async_copy(
                rce_hbm.at[pl.ds(g * 3 * ACHUNK, 3 * ACHUNK)],
                rceb[b], seme[b]).start()

        def wait_edge(b):
            pltpu.make_async_copy(
                rce_hbm.at[pl.ds(0, 3 * ACHUNK)], rceb[b], seme[b]).wait()

        def build(b):
            for j in range(ACHUNK // 16):
                sl = pl.ds(j * 16, 16)
                r16 = rceb[b][sl]
                c16 = rceb[b][pl.ds(ACHUNK + j * 16, 16)]
                w16 = plsc.bitcast(
                    rceb[b][pl.ds(2 * ACHUNK + j * 16, 16)], jnp.float32)
                dr = plsc.load_gather(disv, [r16])
                dc = plsc.load_gather(disv, [c16])
                normb[b][sl] = dr * dc * w16
                gidx[b][sl] = r16 + base_feat
                colb[b][sl] = c16

        def start_gather(b):
            pltpu.make_async_copy(x_hbm.at[gidx[b]], rowsb[b], semg[b]).start()

        def wait_gather(b):
            pltpu.make_async_copy(x_hbm.at[gidx[b]], rowsb[b], semg[b]).wait()

        def start_scatter(b):
            pltpu.make_async_copy(
                rowsb[b], acc.at[colb[b]], sems[b]).start(add=True)

        def wait_scatter(b):
            pltpu.make_async_copy(
                rowsb[b], acc.at[colb[b]], sems[b]).wait()

        for b in range(RING):
            start_edge(b, b)
        for b in range(2):
            wait_edge(b)
            build(b)
            start_gather(b)

        def visit(b, i):
            @pl.when(i < n_w)
            def _():
                wait_gather(b)

                def scale(g, _):
                    n16 = normb[b][pl.ds(g * 16, 16)]
                    for e in range(16):
                        jj = g * 16 + e
                        sn = n16[e]
                        for l in range(HALF // 16):
                            sl = pl.ds(l * 16, 16)
                            rowsb[b][jj, sl] = rowsb[b][jj, sl] * sn
                    return 0
                lax.fori_loop(0, ACHUNK // 16, scale, 0)
                start_scatter(b)
                b2 = (b + 2) % RING

                @pl.when(i + 2 < n_w)
                def _():
                    @pl.when(i >= 2)
                    def _():
                        wait_scatter(b2)
                    wait_edge(b2)
                    build(b2)
                    start_gather(b2)

                @pl.when(i + RING < n_w)
                def _():
                    start_edge(b, i + RING)

        def tbody(t, _):
            for b in range(RING):
                visit(b, t * RING + b)
            return 0
        max_nw = (N_CH + N_SUB - 1) // N_SUB
        lax.fori_loop(0, (max_nw + RING - 1) // RING, tbody, 0)
        for b in range(RING):
            wait_scatter(b)
        plsc.subcore_barrier()

        out_base = base_feat + s * ROWS_PER_SUB
        pltpu.sync_copy(acc.at[pl.ds(s * ROWS_PER_SUB, ROWS_PER_SUB)],
                        out_hbm.at[pl.ds(out_base, ROWS_PER_SUB)])

    f = pl.kernel(
        body,
        out_type=jax.ShapeDtypeStruct((2 * N_PAD, HALF), jnp.float32),
        mesh=_sc_mesh(),
        compiler_params=_SC_PARAMS,
        scratch_types=[
            pltpu.VMEM((N_PAD,), jnp.float32),
        ] + [pltpu.VMEM((3 * ACHUNK,), jnp.int32) for _ in range(RING)]
          + [pltpu.VMEM((ACHUNK,), jnp.int32) for _ in range(RING)]
          + [pltpu.VMEM((ACHUNK,), jnp.int32) for _ in range(RING)]
          + [pltpu.VMEM((ACHUNK,), jnp.float32) for _ in range(RING)]
          + [pltpu.VMEM((ACHUNK, HALF), jnp.float32) for _ in range(RING)]
          + [
            pltpu.VMEM((ZROWS, HALF), jnp.float32),
            pltpu.VMEM_SHARED((N_PAD, HALF), jnp.float32),
        ] + [pltpu.SemaphoreType.DMA for _ in range(3 * RING)],
    )
    return f(rce, dis_pad, xcat)


def _tc_layer1(agg1, x, dis_col, W1, b1r, g1r, be1r, W2):
    def body(agg_ref, x_ref, dis_ref, W1_ref, b1_ref, g1_ref, be1_ref,
             W2_ref, t2_ref):
        d = dis_ref[...]
        pre = agg_ref[...] + d * d * x_ref[...]
        h = jnp.dot(pre, W1_ref[...], preferred_element_type=jnp.float32,
                    precision=lax.Precision.HIGHEST) + b1_ref[...]
        h = jnp.maximum(h, 0.0)
        mu = jnp.mean(h, axis=-1, keepdims=True)
        xc = h - mu
        var = jnp.mean(xc * xc, axis=-1, keepdims=True)
        h = xc * lax.rsqrt(var + 1e-5) * g1_ref[...] + be1_ref[...]
        t2_ref[...] = jnp.dot(h, W2_ref[...], preferred_element_type=jnp.float32,
                              precision=lax.Precision.HIGHEST)

    grid = N_NODES // ROW_BLK
    return pl.pallas_call(
        body,
        grid=(grid,),
        in_specs=[
            pl.BlockSpec((ROW_BLK, 256), lambda i: (i, 0)),
            pl.BlockSpec((ROW_BLK, 256), lambda i: (i, 0)),
            pl.BlockSpec((ROW_BLK, 1), lambda i: (i, 0)),
            pl.BlockSpec((256, 512), lambda i: (0, 0)),
            pl.BlockSpec((1, 512), lambda i: (0, 0)),
            pl.BlockSpec((1, 512), lambda i: (0, 0)),
            pl.BlockSpec((1, 512), lambda i: (0, 0)),
            pl.BlockSpec((512, 256), lambda i: (0, 0)),
        ],
        out_specs=pl.BlockSpec((ROW_BLK, 256), lambda i: (i, 0)),
        out_shape=jax.ShapeDtypeStruct((N_NODES, 256), jnp.float32),
    )(agg1, x, dis_col, W1, b1r, g1r, be1r, W2)


def _tc_layer2_pool(agg2, t2, dis_col, b2r, g2r, be2r, batch_row, fcW, fcb_r):
    def body(agg_ref, t2_ref, dis_ref, b2_ref, g2_ref, be2_ref, bat_ref,
             fcW_ref, fcb_ref, out_ref, pooled, cnt):
        i = pl.program_id(0)

        @pl.when(i == 0)
        def _():
            pooled[...] = jnp.zeros_like(pooled)
            cnt[...] = jnp.zeros_like(cnt)

        d = dis_ref[...]
        pre = agg_ref[...] + d * d * t2_ref[...] + b2_ref[...]
        h = jnp.maximum(pre, 0.0)
        mu = jnp.mean(h, axis=-1, keepdims=True)
        xc = h - mu
        var = jnp.mean(xc * xc, axis=-1, keepdims=True)
        h = xc * lax.rsqrt(var + 1e-5) * g2_ref[...] + be2_ref[...]

        seg = bat_ref[...].reshape(1, ROW_BLK)
        gids = lax.broadcasted_iota(jnp.int32, (N_GRAPHS, ROW_BLK), 0)
        oh = (gids == seg).astype(jnp.float32)
        pooled[...] += jnp.dot(oh, h, preferred_element_type=jnp.float32,
                               precision=lax.Precision.HIGHEST)
        cnt[...] += jnp.sum(oh, axis=1, keepdims=True)

        @pl.when(i == pl.num_programs(0) - 1)
        def _():
            pm = pooled[...] / jnp.maximum(cnt[...], 1.0)
            out_ref[...] = jnp.dot(pm, fcW_ref[...],
                                   preferred_element_type=jnp.float32,
                                   precision=lax.Precision.HIGHEST) + fcb_ref[...]

    grid = N_NODES // ROW_BLK
    return pl.pallas_call(
        body,
        grid=(grid,),
        in_specs=[
            pl.BlockSpec((ROW_BLK, 256), lambda i: (i, 0)),
            pl.BlockSpec((ROW_BLK, 256), lambda i: (i, 0)),
            pl.BlockSpec((ROW_BLK, 1), lambda i: (i, 0)),
            pl.BlockSpec((1, 256), lambda i: (0, 0)),
            pl.BlockSpec((1, 256), lambda i: (0, 0)),
            pl.BlockSpec((1, 256), lambda i: (0, 0)),
            pl.BlockSpec((1, 1, ROW_BLK), lambda i: (i, 0, 0)),
            pl.BlockSpec((256, 128), lambda i: (0, 0)),
            pl.BlockSpec((1, 128), lambda i: (0, 0)),
        ],
        out_specs=pl.BlockSpec((N_GRAPHS, 128), lambda i: (0, 0)),
        out_shape=jax.ShapeDtypeStruct((N_GRAPHS, 128), jnp.float32),
        scratch_shapes=[
            pltpu.VMEM((N_GRAPHS, 256), jnp.float32),
            pltpu.VMEM((N_GRAPHS, 1), jnp.float32),
        ],
    )(agg2, t2, dis_col, b2r, g2r, be2r, batch_row, fcW, fcb_r)


def _split_halves(a):
    ap = jnp.pad(a, ((0, N_PAD - N_NODES), (0, 0)))
    return ap.reshape(N_PAD, 2, HALF).transpose(1, 0, 2).reshape(2 * N_PAD, HALF)


def _merge_halves(a):
    return (a.reshape(2, N_PAD, HALF).transpose(1, 0, 2)
            .reshape(N_PAD, 256)[:N_NODES])


def kernel(x, edge_index, edge_weight, batch,
           W1, b1, g1, be1, W2, b2, g2, be2, fcW, fcb):
    row = edge_index[0].astype(jnp.int32)
    col = edge_index[1].astype(jnp.int32)
    ew = edge_weight.astype(jnp.float32)
    rce = jnp.concatenate(
        [row.reshape(N_CH, ACHUNK), col.reshape(N_CH, ACHUNK),
         lax.bitcast_convert_type(ew, jnp.int32).reshape(N_CH, ACHUNK)],
        axis=1).reshape(-1)

    deg_sc = _sc_deg(col.reshape(N_SUB, DCPW, DCHUNK),
                     ew.reshape(N_SUB, DCPW, DCHUNK))
    deg = deg_sc[:N_NODES] + 1.0
    dis = jnp.where(deg > 0, lax.rsqrt(deg), 0.0)
    dis_pad = jnp.pad(dis, (0, N_PAD - N_NODES))
    dis_col = dis[:, None]

    agg1 = _merge_halves(_sc_agg(rce, dis_pad, _split_halves(x)))
    t2 = _tc_layer1(agg1, x, dis_col, W1, b1[None], g1[None], be1[None], W2)
    agg2 = _merge_halves(_sc_agg(rce, dis_pad, _split_halves(t2)))
    return _tc_layer2_pool(agg2, t2, dis_col, b2[None], g2[None], be2[None],
                           batch.astype(jnp.int32).reshape(
                               N_NODES // ROW_BLK, 1, ROW_BLK),
                           fcW, fcb[None])


# speedup vs baseline: 19.3535x; 1.2363x over previous
import jax
import jax.numpy as jnp
from jax import lax
from jax.experimental import pallas as pl
from jax.experimental.pallas import tpu as pltpu
from jax.experimental.pallas import tpu_sc as plsc

N_NODES = 10000
N_PAD = 10240
N_EDGES = 160000
HALF = 128
N_SUB = 16
N_CORES = 2
DCHUNK = 80
DCPW = (N_EDGES // DCHUNK) // N_SUB
ACHUNK = 64
N_CH = N_EDGES // ACHUNK
RING = 4
ROWS_PER_SUB = N_PAD // N_SUB
ZROWS = 16
N_GRAPHS = 64
ROW_BLK = 2000


def _sc_mesh():
    return plsc.VectorSubcoreMesh(core_axis_name="c", subcore_axis_name="s")


_SC_PARAMS = pltpu.CompilerParams(needs_layout_passes=False)


def _sc_deg(col2, ew2):
    def body(col_hbm, ew_hbm, deg_hbm, colv2, eww2, zbuf, acc, sem):
        c = lax.axis_index("c")
        s = lax.axis_index("s")

        @pl.when(c == 0)
        def _():
            def zb(i, _):
                zbuf[pl.ds(i * 16, 16)] = jnp.zeros((16,), jnp.float32)
                return 0
            lax.fori_loop(0, ROWS_PER_SUB // 16, zb, 0)
            pltpu.sync_copy(zbuf, acc.at[pl.ds(s * ROWS_PER_SUB, ROWS_PER_SUB)])
            plsc.subcore_barrier()

            pltpu.sync_copy(col_hbm.at[s], colv2)
            pltpu.sync_copy(ew_hbm.at[s], eww2)

            def step(t, _):
                for b in range(5):
                    ci = t * 5 + b
                    pltpu.make_async_copy(
                        eww2.at[ci], acc.at[colv2.at[ci]], sem).start(add=True)
                for b in range(5):
                    pltpu.make_async_copy(
                        eww2.at[0], acc.at[colv2.at[0]], sem).wait()
                return 0
            lax.fori_loop(0, DCPW // 5, step, 0)
            plsc.subcore_barrier()
            pltpu.sync_copy(acc.at[pl.ds(s * ROWS_PER_SUB, ROWS_PER_SUB)],
                            deg_hbm.at[pl.ds(s * ROWS_PER_SUB, ROWS_PER_SUB)])

    f = pl.kernel(
        body,
        out_type=jax.ShapeDtypeStruct((N_PAD,), jnp.float32),
        mesh=_sc_mesh(),
        compiler_params=_SC_PARAMS,
        scratch_types=[
            pltpu.VMEM((DCPW, DCHUNK), jnp.int32),
            pltpu.VMEM((DCPW, DCHUNK), jnp.float32),
            pltpu.VMEM((ROWS_PER_SUB,), jnp.float32),
            pltpu.VMEM_SHARED((N_PAD,), jnp.float32),
            pltpu.SemaphoreType.DMA,
        ],
    )
    return f(col2, ew2)


def _sc_agg(rce, dis_pad, xcat, nx):
    def body(rce_hbm, dis_hbm, x_hbm, out_hbm,
             disv,
             rce0, rce1, rce2, rce3,
             colb0, colb1, colb2, colb3,
             gidx0, gidx1, gidx2, gidx3,
             norm0, norm1, norm2, norm3,
             rows0, rows1, rows2, rows3,
             zbuf, acc,
             seme0, seme1, seme2, seme3,
             semg0, semg1, semg2, semg3,
             sems0, sems1, sems2, sems3):
        c = lax.axis_index("c")
        s = lax.axis_index("s")
        base_feat = c * nx
        rceb = [rce0, rce1, rce2, rce3]
        colb = [colb0, colb1, colb2, colb3]
        gidx = [gidx0, gidx1, gidx2, gidx3]
        normb = [norm0, norm1, norm2, norm3]
        rowsb = [rows0, rows1, rows2, rows3]
        seme = [seme0, seme1, seme2, seme3]
        semg = [semg0, semg1, semg2, semg3]
        sems = [sems0, sems1, sems2, sems3]

        pltpu.sync_copy(dis_hbm, disv)

        def zb(r, _):
            for l in range(HALF // 16):
                zbuf[r, pl.ds(l * 16, 16)] = jnp.zeros((16,), jnp.float32)
            return 0
        lax.fori_loop(0, ZROWS, zb, 0)

        def zc(t, _):
            pltpu.sync_copy(
                zbuf, acc.at[pl.ds(s * ROWS_PER_SUB + t * ZROWS, ZROWS)])
            return 0
        lax.fori_loop(0, ROWS_PER_SUB // ZROWS, zc, 0)
        plsc.subcore_barrier()

        n_w = (N_CH - s + N_SUB - 1) // N_SUB

        def start_edge(b, i):
            g = s + N_SUB * i
            pltpu.make_async_copy(
                rce_hbm.at[pl.ds(g * 3 * ACHUNK, 3 * ACHUNK)],
                rceb[b], seme[b]).start()

        def wait_edge(b):
            pltpu.make_async_copy(
                rce_hbm.at[pl.ds(0, 3 * ACHUNK)], rceb[b], seme[b]).wait()

        def build(b):
            for j in range(ACHUNK // 16):
                sl = pl.ds(j * 16, 16)
                r16 = rceb[b][sl]
                c16 = rceb[b][pl.ds(ACHUNK + j * 16, 16)]
                w16 = plsc.bitcast(
                    rceb[b][pl.ds(2 * ACHUNK + j * 16, 16)], jnp.float32)
                dr = plsc.load_gather(disv, [r16])
                dc = plsc.load_gather(disv, [c16])
                normb[b][sl] = dr * dc * w16
                gidx[b][sl] = r16 + base_feat
                colb[b][sl] = c16

        def start_gather(b):
            pltpu.make_async_copy(x_hbm.at[gidx[b]], rowsb[b], semg[b]).start()

        def wait_gather(b):
            pltpu.make_async_copy(x_hbm.at[gidx[b]], rowsb[b], semg[b]).wait()

        def start_scatter(b):
            pltpu.make_async_copy(
                rowsb[b], acc.at[colb[b]], sems[b]).start(add=True)

        def wait_scatter(b):
            pltpu.make_async_copy(
                rowsb[b], acc.at[colb[b]], sems[b]).wait()

        for b in range(RING):
            start_edge(b, b)
        for b in range(2):
            wait_edge(b)
            build(b)
            start_gather(b)

        def visit(b, i):
            @pl.when(i < n_w)
            def _():
                wait_gather(b)

                def scale(g, _):
                    n16 = normb[b][pl.ds(g * 16, 16)]
                    for e in range(16):
                        jj = g * 16 + e
                        sn = n16[e]
                        for l in range(HALF // 16):
                            sl = pl.ds(l * 16, 16)
                            rowsb[b][jj, sl] = rowsb[b][jj, sl] * sn
                    return 0
                lax.fori_loop(0, ACHUNK // 16, scale, 0)
                start_scatter(b)
                b2 = (b + 2) % RING

                @pl.when(i + 2 < n_w)
                def _():
                    @pl.when(i >= 2)
                    def _():
                        wait_scatter(b2)
                    wait_edge(b2)
                    build(b2)
                    start_gather(b2)

                @pl.when(i + RING < n_w)
                def _():
                    start_edge(b, i + RING)

        def tbody(t, _):
            for b in range(RING):
                visit(b, t * RING + b)
            return 0
        max_nw = (N_CH + N_SUB - 1) // N_SUB
        lax.fori_loop(0, (max_nw + RING - 1) // RING, tbody, 0)
        for b in range(RING):
            wait_scatter(b)
        plsc.subcore_barrier()

        out_base = base_feat + s * ROWS_PER_SUB
        last = nx - (N_SUB - 1) * ROWS_PER_SUB

        @pl.when(s < N_SUB - 1)
        def _():
            pltpu.sync_copy(acc.at[pl.ds(s * ROWS_PER_SUB, ROWS_PER_SUB)],
                            out_hbm.at[pl.ds(out_base, ROWS_PER_SUB)])

        @pl.when(s == N_SUB - 1)
        def _():
            pltpu.sync_copy(acc.at[pl.ds(s * ROWS_PER_SUB, last)],
                            out_hbm.at[pl.ds(out_base, last)])

    f = pl.kernel(
        body,
        out_type=jax.ShapeDtypeStruct((2 * nx, HALF), jnp.float32),
        mesh=_sc_mesh(),
        compiler_params=_SC_PARAMS,
        scratch_types=[
            pltpu.VMEM((N_PAD,), jnp.float32),
        ] + [pltpu.VMEM((3 * ACHUNK,), jnp.int32) for _ in range(RING)]
          + [pltpu.VMEM((ACHUNK,), jnp.int32) for _ in range(RING)]
          + [pltpu.VMEM((ACHUNK,), jnp.int32) for _ in range(RING)]
          + [pltpu.VMEM((ACHUNK,), jnp.float32) for _ in range(RING)]
          + [pltpu.VMEM((ACHUNK, HALF), jnp.float32) for _ in range(RING)]
          + [
            pltpu.VMEM((ZROWS, HALF), jnp.float32),
            pltpu.VMEM_SHARED((N_PAD, HALF), jnp.float32),
        ] + [pltpu.SemaphoreType.DMA for _ in range(3 * RING)],
    )
    return f(rce, dis_pad, xcat)


def _tc_layer1(aggc, x, dis_col, W1, b1r, g1r, be1r, W2a, W2b):
    def body(lo_ref, hi_ref, x_ref, dis_ref, W1_ref, b1_ref, g1_ref,
             be1_ref, W2a_ref, W2b_ref, t2_ref, tc_ref):
        d = dis_ref[...]
        agg = jnp.concatenate([lo_ref[...], hi_ref[...]], axis=-1)
        pre = agg + d * d * x_ref[...]
        h = jnp.dot(pre, W1_ref[...], preferred_element_type=jnp.float32,
                    precision=lax.Precision.HIGHEST) + b1_ref[...]
        h = jnp.maximum(h, 0.0)
        mu = jnp.mean(h, axis=-1, keepdims=True)
        xc = h - mu
        var = jnp.mean(xc * xc, axis=-1, keepdims=True)
        h = xc * lax.rsqrt(var + 1e-5) * g1_ref[...] + be1_ref[...]
        ta = jnp.dot(h, W2a_ref[...], preferred_element_type=jnp.float32,
                     precision=lax.Precision.HIGHEST)
        tb = jnp.dot(h, W2b_ref[...], preferred_element_type=jnp.float32,
                     precision=lax.Precision.HIGHEST)
        t2_ref[...] = jnp.concatenate([ta, tb], axis=-1)
        tc_ref[0, :, :] = ta
        tc_ref[1, :, :] = tb

    grid = N_NODES // ROW_BLK
    return pl.pallas_call(
        body,
        grid=(grid,),
        in_specs=[
            pl.BlockSpec((ROW_BLK, HALF), lambda i: (i, 0)),
            pl.BlockSpec((ROW_BLK, HALF), lambda i: (grid + i, 0)),
            pl.BlockSpec((ROW_BLK, 256), lambda i: (i, 0)),
            pl.BlockSpec((ROW_BLK, 1), lambda i: (i, 0)),
            pl.BlockSpec((256, 512), lambda i: (0, 0)),
            pl.BlockSpec((1, 512), lambda i: (0, 0)),
            pl.BlockSpec((1, 512), lambda i: (0, 0)),
            pl.BlockSpec((1, 512), lambda i: (0, 0)),
            pl.BlockSpec((512, HALF), lambda i: (0, 0)),
            pl.BlockSpec((512, HALF), lambda i: (0, 0)),
        ],
        out_specs=[
            pl.BlockSpec((ROW_BLK, 256), lambda i: (i, 0)),
            pl.BlockSpec((2, ROW_BLK, HALF), lambda i: (0, i, 0)),
        ],
        out_shape=[
            jax.ShapeDtypeStruct((N_NODES, 256), jnp.float32),
            jax.ShapeDtypeStruct((2, N_NODES, HALF), jnp.float32),
        ],
    )(aggc, aggc, x, dis_col, W1, b1r, g1r, be1r, W2a, W2b)


def _tc_layer2_pool(aggc, t2, dis_col, b2r, g2r, be2r, batch_row, fcW, fcb_r):
    def body(lo_ref, hi_ref, t2_ref, dis_ref, b2_ref, g2_ref, be2_ref,
             bat_ref, fcW_ref, fcb_ref, out_ref, pooled, cnt):
        i = pl.program_id(0)

        @pl.when(i == 0)
        def _():
            pooled[...] = jnp.zeros_like(pooled)
            cnt[...] = jnp.zeros_like(cnt)

        d = dis_ref[...]
        agg = jnp.concatenate([lo_ref[...], hi_ref[...]], axis=-1)
        pre = agg + d * d * t2_ref[...] + b2_ref[...]
        h = jnp.maximum(pre, 0.0)
        mu = jnp.mean(h, axis=-1, keepdims=True)
        xc = h - mu
        var = jnp.mean(xc * xc, axis=-1, keepdims=True)
        h = xc * lax.rsqrt(var + 1e-5) * g2_ref[...] + be2_ref[...]

        seg = bat_ref[...].reshape(1, ROW_BLK)
        gids = lax.broadcasted_iota(jnp.int32, (N_GRAPHS, ROW_BLK), 0)
        oh = (gids == seg).astype(jnp.float32)
        pooled[...] += jnp.dot(oh, h, preferred_element_type=jnp.float32,
                               precision=lax.Precision.HIGHEST)
        cnt[...] += jnp.sum(oh, axis=1, keepdims=True)

        @pl.when(i == pl.num_programs(0) - 1)
        def _():
            pm = pooled[...] / jnp.maximum(cnt[...], 1.0)
            out_ref[...] = jnp.dot(pm, fcW_ref[...],
                                   preferred_element_type=jnp.float32,
                                   precision=lax.Precision.HIGHEST) + fcb_ref[...]

    grid = N_NODES // ROW_BLK
    return pl.pallas_call(
        body,
        grid=(grid,),
        in_specs=[
            pl.BlockSpec((ROW_BLK, HALF), lambda i: (i, 0)),
            pl.BlockSpec((ROW_BLK, HALF), lambda i: (grid + i, 0)),
            pl.BlockSpec((ROW_BLK, 256), lambda i: (i, 0)),
            pl.BlockSpec((ROW_BLK, 1), lambda i: (i, 0)),
            pl.BlockSpec((1, 256), lambda i: (0, 0)),
            pl.BlockSpec((1, 256), lambda i: (0, 0)),
            pl.BlockSpec((1, 256), lambda i: (0, 0)),
            pl.BlockSpec((1, 1, ROW_BLK), lambda i: (i, 0, 0)),
            pl.BlockSpec((256, 128), lambda i: (0, 0)),
            pl.BlockSpec((1, 128), lambda i: (0, 0)),
        ],
        out_specs=pl.BlockSpec((N_GRAPHS, 128), lambda i: (0, 0)),
        out_shape=jax.ShapeDtypeStruct((N_GRAPHS, 128), jnp.float32),
        scratch_shapes=[
            pltpu.VMEM((N_GRAPHS, 256), jnp.float32),
            pltpu.VMEM((N_GRAPHS, 1), jnp.float32),
        ],
    )(aggc, aggc, t2, dis_col, b2r, g2r, be2r, batch_row, fcW, fcb_r)


def _split_halves(a):
    return (a.reshape(N_NODES, 2, HALF).transpose(1, 0, 2)
            .reshape(2 * N_NODES, HALF))


def kernel(x, edge_index, edge_weight, batch,
           W1, b1, g1, be1, W2, b2, g2, be2, fcW, fcb):
    row = edge_index[0].astype(jnp.int32)
    col = edge_index[1].astype(jnp.int32)
    ew = edge_weight.astype(jnp.float32)
    rce = jnp.concatenate(
        [row.reshape(N_CH, ACHUNK), col.reshape(N_CH, ACHUNK),
         lax.bitcast_convert_type(ew, jnp.int32).reshape(N_CH, ACHUNK)],
        axis=1).reshape(-1)

    deg_sc = _sc_deg(col.reshape(N_SUB, DCPW, DCHUNK),
                     ew.reshape(N_SUB, DCPW, DCHUNK))
    deg = deg_sc[:N_NODES] + 1.0
    dis = jnp.where(deg > 0, lax.rsqrt(deg), 0.0)
    dis_pad = jnp.pad(dis, (0, N_PAD - N_NODES))
    dis_col = dis[:, None]

    agg1c = _sc_agg(rce, dis_pad, _split_halves(x), N_NODES)
    t2, t2cat = _tc_layer1(agg1c, x, dis_col, W1, b1[None], g1[None],
                           be1[None], W2[:, :HALF], W2[:, HALF:])
    agg2c = _sc_agg(rce, dis_pad, t2cat.reshape(2 * N_NODES, HALF), N_NODES)
    return _tc_layer2_pool(agg2c, t2, dis_col, b2[None], g2[None], be2[None],
                           batch.astype(jnp.int32).reshape(
                               N_NODES // ROW_BLK, 1, ROW_BLK),
                           fcW, fcb[None])
